# baseline (speedup 1.0000x reference)
import jax
import jax.numpy as jnp
from jax.experimental import pallas as pl
from jax.experimental.pallas import tpu as pltpu

N = 10000
D = 128
C = 16
CH = 1000
NCH = N // CH
NBUF = 3

_SELU_ALPHA = 1.6732632423543772848170429916717
_SELU_SCALE = 1.0507009873554804934193349852946


def _chain(x, w, b2):
    logits = jnp.dot(x, w, preferred_element_type=jnp.float32)
    lt = logits.T + b2
    m = jnp.max(lt, axis=0, keepdims=True)
    e = jnp.exp(lt - m)
    at = e / jnp.sum(e, axis=0, keepdims=True)
    return at, jnp.sum(at, axis=1, keepdims=True)


def _dmon_kernel(x_hbm, w_ref, b_ref, pooled_ref, assign_hbm,
                 xbuf, obuf, in_sems, out_sems):
    w = w_ref[...]
    b2 = b_ref[...]

    def start_in(j):
        pltpu.make_async_copy(
            x_hbm.at[pl.ds(j * CH, CH), :], xbuf.at[j % NBUF], in_sems.at[j % NBUF]
        ).start()

    def wait_in(j):
        pltpu.make_async_copy(
            x_hbm.at[pl.ds(j * CH, CH), :], xbuf.at[j % NBUF], in_sems.at[j % NBUF]
        ).wait()

    def start_out(j):
        pltpu.make_async_copy(
            obuf.at[j % NBUF], assign_hbm.at[pl.ds(j * CH, CH), :],
            out_sems.at[j % NBUF],
        ).start()

    def wait_out(j):
        pltpu.make_async_copy(
            obuf.at[j % NBUF], assign_hbm.at[pl.ds(j * CH, CH), :],
            out_sems.at[j % NBUF],
        ).wait()

    for j in range(min(NBUF, NCH)):
        start_in(j)

    pooled = jnp.zeros((C, D), jnp.float32)
    sizes = jnp.zeros((C, 1), jnp.float32)

    for j in range(NCH):
        p = j % NBUF
        wait_in(j)
        x = xbuf[p]
        at, s = _chain(x, w, b2)
        if j >= NBUF:
            wait_out(j - NBUF)
        obuf[p] = at.T
        start_out(j)
        if j + NBUF < NCH:
            start_in(j + NBUF)
        pooled = pooled + jax.lax.dot_general(
            at, x, (((1,), (0,)), ((), ())),
            preferred_element_type=jnp.float32,
        )
        sizes = sizes + s

    for j in range(max(NCH - NBUF, 0), NCH):
        wait_out(j)

    pooled = pooled / sizes
    pooled_ref[...] = _SELU_SCALE * jnp.where(
        pooled > 0, pooled, _SELU_ALPHA * (jnp.exp(pooled) - 1.0)
    )


def kernel(features, edge_index, W, b):
    del edge_index
    b2 = b.reshape(C, 1)
    features_pooled, assignments = pl.pallas_call(
        _dmon_kernel,
        in_specs=[
            pl.BlockSpec(memory_space=pltpu.MemorySpace.HBM),
            pl.BlockSpec(memory_space=pltpu.MemorySpace.VMEM),
            pl.BlockSpec(memory_space=pltpu.MemorySpace.VMEM),
        ],
        out_specs=[
            pl.BlockSpec(memory_space=pltpu.MemorySpace.VMEM),
            pl.BlockSpec(memory_space=pltpu.MemorySpace.HBM),
        ],
        out_shape=[
            jax.ShapeDtypeStruct((C, D), jnp.float32),
            jax.ShapeDtypeStruct((N, C), jnp.float32),
        ],
        scratch_shapes=[
            pltpu.VMEM((NBUF, CH, D), jnp.float32),
            pltpu.VMEM((NBUF, CH, C), jnp.float32),
            pltpu.SemaphoreType.DMA((NBUF,)),
            pltpu.SemaphoreType.DMA((NBUF,)),
        ],
    )(features, W, b2)
    return (features_pooled, assignments)


# speedup vs baseline: 1.1987x; 1.1987x over previous
import jax
import jax.numpy as jnp
from jax.experimental import pallas as pl
from jax.experimental.pallas import tpu as pltpu

N = 10000
D = 128
C = 16
K = 5
BNW = 1000
GRID = N // (K * BNW)

_SELU_ALPHA = 1.6732632423543772848170429916717
_SELU_SCALE = 1.0507009873554804934193349852946


def _chain(x, w, b2):
    logits = jnp.dot(x, w, preferred_element_type=jnp.float32)
    lt = logits.T + b2
    m = jnp.max(lt, axis=0, keepdims=True)
    e = jnp.exp(lt - m)
    at = e / jnp.sum(e, axis=0, keepdims=True)
    return at, jnp.sum(at, axis=1, keepdims=True)


def _dmon_kernel(x0_ref, x1_ref, x2_ref, x3_ref, x4_ref, w_ref, b_ref,
                 pooled_ref, assign_ref, s_ref):
    i = pl.program_id(0)
    w = w_ref[...]
    b2 = b_ref[...]
    xs = [x0_ref[...], x1_ref[...], x2_ref[...], x3_ref[...], x4_ref[...]]

    part = None
    part_s = None
    for j, x in enumerate(xs):
        at, s = _chain(x, w, b2)
        assign_ref[j * BNW:(j + 1) * BNW, :] = at.T
        p = jax.lax.dot_general(
            at, x, (((1,), (0,)), ((), ())),
            preferred_element_type=jnp.float32,
        )
        part = p if part is None else part + p
        part_s = s if part_s is None else part_s + s

    @pl.when(i == 0)
    def _init():
        pooled_ref[...] = part
        s_ref[...] = part_s

    @pl.when(i > 0)
    def _acc():
        pooled_ref[...] += part
        s_ref[...] += part_s

    @pl.when(i == GRID - 1)
    def _finalize():
        pooled = pooled_ref[...] / s_ref[...]
        pooled_ref[...] = _SELU_SCALE * jnp.where(
            pooled > 0, pooled, _SELU_ALPHA * (jnp.exp(pooled) - 1.0)
        )


def kernel(features, edge_index, W, b):
    del edge_index
    b2 = b.reshape(C, 1)

    def xspec(j):
        return pl.BlockSpec((BNW, D), lambda i, j=j: (K * i + j, 0))

    features_pooled, assignments = pl.pallas_call(
        _dmon_kernel,
        grid=(GRID,),
        in_specs=[
            xspec(0), xspec(1), xspec(2), xspec(3), xspec(4),
            pl.BlockSpec((D, C), lambda i: (0, 0)),
            pl.BlockSpec((C, 1), lambda i: (0, 0)),
        ],
        out_specs=[
            pl.BlockSpec((C, D), lambda i: (0, 0)),
            pl.BlockSpec((K * BNW, C), lambda i: (i, 0)),
        ],
        out_shape=[
            jax.ShapeDtypeStruct((C, D), jnp.float32),
            jax.ShapeDtypeStruct((N, C), jnp.float32),
        ],
        scratch_shapes=[pltpu.VMEM((C, 1), jnp.float32)],
        compiler_params=pltpu.CompilerParams(
            dimension_semantics=("arbitrary",),
        ),
    )(features, features, features, features, features, W, b2)
    return (features_pooled, assignments)
